# baseline (speedup 1.0000x reference)
import jax
import jax.numpy as jnp
from jax.experimental import pallas as pl
from jax.experimental.pallas import tpu as pltpu

_OLD_W, _OLD_H = 224, 224
_NEW_W, _NEW_H = 28, 28
_CELLS = _NEW_W * _NEW_H
_BG = 8


def _scatter_kernel(p_ref, v_ref, enc_ref, out_ref):
    g = pl.program_id(0)
    out_ref[...] = jnp.zeros(out_ref.shape, out_ref.dtype)
    n_sv = enc_ref.shape[0]
    for n in range(n_sv):
        updates = []
        for b8 in range(_BG):
            pn = p_ref[g * _BG + b8, n]
            vn = v_ref[g * _BG + b8, n].astype(jnp.float32)
            row = enc_ref[n, b8 : b8 + 1, :] * vn
            updates.append((pn, jnp.maximum(out_ref[pn, b8 : b8 + 1, :], row)))
        for b8 in range(_BG):
            pn, val = updates[b8]
            out_ref[pn, b8 : b8 + 1, :] = val


def kernel(svPositionsAtT0, svEncoding, lengths):
    b_, n_, hid = svEncoding.shape
    x = svPositionsAtT0[..., 0]
    y = svPositionsAtT0[..., 1]
    xIdx = (x * _NEW_W / _OLD_W).astype(jnp.int32)
    yIdx = (y * _NEW_H / _OLD_H).astype(jnp.int32)
    agent_ids = jnp.arange(n_, dtype=lengths.dtype)[None, :]
    valid = (lengths[:, None] > agent_ids) & (xIdx < _NEW_W) & (yIdx < _NEW_H)
    xI = jnp.clip(xIdx, 0, _NEW_W - 1)
    yI = jnp.clip(yIdx, 0, _NEW_H - 1)
    p = (xI * _NEW_H + yI).astype(jnp.int32)
    v = valid.astype(jnp.int32)

    enc_t = svEncoding.transpose(1, 0, 2)

    zz = pl.pallas_call(
        _scatter_kernel,
        grid_spec=pltpu.PrefetchScalarGridSpec(
            num_scalar_prefetch=2,
            grid=(b_ // _BG, 2),
            in_specs=[
                pl.BlockSpec((n_, _BG, hid // 2), lambda g, h, pr, vr: (0, g, h)),
            ],
            out_specs=pl.BlockSpec(
                (_CELLS, _BG, hid // 2), lambda g, h, pr, vr: (0, g, h)
            ),
        ),
        out_shape=jax.ShapeDtypeStruct((_CELLS, b_, hid), jnp.float32),
        compiler_params=pltpu.CompilerParams(
            dimension_semantics=("arbitrary", "arbitrary")
        ),
    )(p, v, enc_t)
    return zz.reshape(_NEW_W, _NEW_H, b_, hid).transpose(2, 3, 0, 1)


# speedup vs baseline: 1.0541x; 1.0541x over previous
import jax
import jax.numpy as jnp
from jax.experimental import pallas as pl
from jax.experimental.pallas import tpu as pltpu

_OLD_W, _OLD_H = 224, 224
_NEW_W, _NEW_H = 28, 28
_CELLS = _NEW_W * _NEW_H
_BG = 8


def _scatter_kernel(p_ref, v_ref, enc_ref, out_ref):
    g = pl.program_id(0)
    out_ref[...] = jnp.zeros(out_ref.shape, out_ref.dtype)
    n_sv = enc_ref.shape[0]
    for n in range(n_sv):
        updates = []
        for b8 in range(_BG):
            pn = p_ref[g * _BG + b8, n]
            vn = v_ref[g * _BG + b8, n].astype(jnp.float32)
            row = enc_ref[n, b8 : b8 + 1, :] * vn
            updates.append((pn, jnp.maximum(out_ref[pn, b8 : b8 + 1, :], row)))
        for b8 in range(_BG):
            pn, val = updates[b8]
            out_ref[pn, b8 : b8 + 1, :] = val


def kernel(svPositionsAtT0, svEncoding, lengths):
    b_, n_, hid = svEncoding.shape
    x = svPositionsAtT0[..., 0]
    y = svPositionsAtT0[..., 1]
    xIdx = (x * _NEW_W / _OLD_W).astype(jnp.int32)
    yIdx = (y * _NEW_H / _OLD_H).astype(jnp.int32)
    agent_ids = jnp.arange(n_, dtype=lengths.dtype)[None, :]
    valid = (lengths[:, None] > agent_ids) & (xIdx < _NEW_W) & (yIdx < _NEW_H)
    xI = jnp.clip(xIdx, 0, _NEW_W - 1)
    yI = jnp.clip(yIdx, 0, _NEW_H - 1)
    p = (xI * _NEW_H + yI).astype(jnp.int32)
    v = valid.astype(jnp.int32)

    enc_t = svEncoding.transpose(1, 0, 2)

    zz = pl.pallas_call(
        _scatter_kernel,
        grid_spec=pltpu.PrefetchScalarGridSpec(
            num_scalar_prefetch=2,
            grid=(b_ // _BG,),
            in_specs=[
                pl.BlockSpec((n_, _BG, hid), lambda g, pr, vr: (0, g, 0)),
            ],
            out_specs=pl.BlockSpec((_CELLS, _BG, hid), lambda g, pr, vr: (0, g, 0)),
        ),
        out_shape=jax.ShapeDtypeStruct((_CELLS, b_, hid), jnp.float32),
        compiler_params=pltpu.CompilerParams(dimension_semantics=("arbitrary",)),
    )(p, v, enc_t)
    return zz.reshape(_NEW_W, _NEW_H, b_, hid).transpose(2, 3, 0, 1)


# speedup vs baseline: 1.1087x; 1.0517x over previous
import jax
import jax.numpy as jnp
from jax.experimental import pallas as pl
from jax.experimental.pallas import tpu as pltpu

_OLD_W, _OLD_H = 224, 224
_NEW_W, _NEW_H = 28, 28
_CELLS = _NEW_W * _NEW_H
_BG = 8


def _scatter_kernel(w_ref, enc_ref, out_ref):
    g = pl.program_id(0)
    out_ref[...] = jnp.zeros(out_ref.shape, out_ref.dtype)
    n_sv = enc_ref.shape[0]
    for n in range(n_sv):
        updates = []
        for b8 in range(_BG):
            wn = w_ref[g * _BG + b8, n]
            pn = wn & (1024 - 1)
            vn = (wn >> 10).astype(jnp.float32)
            row = enc_ref[n, b8 : b8 + 1, :] * vn
            updates.append((pn, jnp.maximum(out_ref[pn, b8 : b8 + 1, :], row)))
        for b8 in range(_BG):
            pn, val = updates[b8]
            out_ref[pn, b8 : b8 + 1, :] = val


def kernel(svPositionsAtT0, svEncoding, lengths):
    b_, n_, hid = svEncoding.shape
    x = svPositionsAtT0[..., 0]
    y = svPositionsAtT0[..., 1]
    xIdx = (x * _NEW_W / _OLD_W).astype(jnp.int32)
    yIdx = (y * _NEW_H / _OLD_H).astype(jnp.int32)
    agent_ids = jnp.arange(n_, dtype=lengths.dtype)[None, :]
    valid = (lengths[:, None] > agent_ids) & (xIdx < _NEW_W) & (yIdx < _NEW_H)
    xI = jnp.clip(xIdx, 0, _NEW_W - 1)
    yI = jnp.clip(yIdx, 0, _NEW_H - 1)
    w = ((xI * _NEW_H + yI) | (valid.astype(jnp.int32) << 10)).astype(jnp.int32)

    enc_t = svEncoding.transpose(1, 0, 2)

    zz = pl.pallas_call(
        _scatter_kernel,
        grid_spec=pltpu.PrefetchScalarGridSpec(
            num_scalar_prefetch=1,
            grid=(b_ // _BG,),
            in_specs=[
                pl.BlockSpec((n_, _BG, hid), lambda g, wr: (0, g, 0)),
            ],
            out_specs=pl.BlockSpec((_CELLS, _BG, hid), lambda g, wr: (0, g, 0)),
        ),
        out_shape=jax.ShapeDtypeStruct((_CELLS, b_, hid), jnp.float32),
        compiler_params=pltpu.CompilerParams(dimension_semantics=("arbitrary",)),
    )(w, enc_t)
    return zz.reshape(_NEW_W, _NEW_H, b_, hid).transpose(2, 3, 0, 1)
